# baseline (speedup 1.0000x reference)
import functools

import jax
import jax.numpy as jnp
from jax import lax
from jax.experimental import pallas as pl
from jax.experimental.pallas import tpu as pltpu
from jax.experimental.pallas import tpu_sc as plsc

NC = 2
NS = 16
NW = NC * NS
D = 32
B = 16384
B_PER_W = B // NW
CH = 128
N_CH = B_PER_W // CH
N_STREAM = 4 * N_CH


def _make_sc_call():
    mesh = plsc.VectorSubcoreMesh(
        core_axis_name="c", subcore_axis_name="s",
        num_cores=NC, num_subcores=NS)

    @functools.partial(
        pl.kernel,
        mesh=mesh,
        compiler_params=pltpu.CompilerParams(use_tc_tiling_on_sc=False),
        out_type=(
            jax.ShapeDtypeStruct((B, 2 * D), jnp.float32),
            jax.ShapeDtypeStruct((B, 2 * D), jnp.float32),
        ),
        scratch_types=(
            [pltpu.VMEM((CH,), jnp.int32) for _ in range(N_STREAM)]
            + [pltpu.VMEM((CH, D), jnp.float32) for _ in range(N_STREAM)]
            + [pltpu.SemaphoreType.DMA]
        ),
    )
    def sc_embed(idx_hbm, W1, W2, W3, out0, out1, *scratch):
        idx_vs = scratch[:N_STREAM]
        bufs = scratch[N_STREAM:2 * N_STREAM]
        sem = scratch[-1]
        wid = lax.axis_index("s") * NC + lax.axis_index("c")
        base = wid * B_PER_W
        tables = (W1, W2, W2, W3)
        outs = (out0, out0, out1, out1)
        for t in range(4):
            for j in range(N_CH):
                pltpu.sync_copy(idx_hbm.at[wid, t, j], idx_vs[t * N_CH + j])
        copies = []
        for t in range(4):
            for j in range(N_CH):
                s = t * N_CH + j
                copies.append(
                    pltpu.async_copy(tables[t].at[idx_vs[s]], bufs[s], sem))
        for t in range(4):
            for j in range(N_CH):
                s = t * N_CH + j
                copies[s].wait()
                pltpu.sync_copy(
                    bufs[s],
                    outs[t].at[pl.ds(base + j * CH, CH),
                               pl.ds((t % 2) * D, D)])

    return sc_embed


_sc_embed = _make_sc_call()


def kernel(Xs_0, Xs_1, W_cat1, W_cat2, W_cat3):
    cols = jnp.stack(
        [Xs_0[:, 0], Xs_0[:, 1], Xs_1[:, 0], Xs_1[:, 1]], axis=0
    ).astype(jnp.int32)
    idx = cols.reshape(4, NW, N_CH, CH).transpose(1, 0, 2, 3)
    out0, out1 = _sc_embed(idx, W_cat1, W_cat2, W_cat3)
    return (out0, out1)
